# baseline (speedup 1.0000x reference)
import functools

import jax
import jax.numpy as jnp
from jax import lax
from jax.experimental import pallas as pl
from jax.experimental.pallas import tpu as pltpu
from jax.experimental.pallas import tpu_sc as plsc

N = 10000
M = 20000
DEG = 8
H = 128
CK = 8
CS = 4
DEPTH = 5
B = 64
IN = H + CK + CS

NC, NS = 2, 16
NW = NC * NS
N_PAD = 10240
M_PAD = 20480


def _sc_gather(table, idx, chunk, nbuf=4, heavy_frac=0.5, heavy_core=0):
    (K,) = idx.shape
    T, D = table.shape
    dt = table.dtype
    cs = K // (NS * chunk)
    assert K % (NS * chunk) == 0 and chunk % 8 == 0 and chunk <= 128
    a_heavy = max(1, min(cs, round(cs * heavy_frac)))
    a_light = cs - a_heavy
    n0, n1 = ((a_heavy, a_light) if heavy_core == 0
              else (a_light, a_heavy))
    nbuf = min(nbuf, max(n0, n1))

    mesh = plsc.VectorSubcoreMesh(core_axis_name="c", subcore_axis_name="s",
                                  num_cores=NC, num_subcores=NS)
    scratch = [pltpu.VMEM((max(n0, n1) * chunk,), jnp.int32)]
    scratch += [pltpu.VMEM((chunk, D), dt) for _ in range(nbuf)]
    scratch += [pltpu.SemaphoreType.DMA for _ in range(2 * nbuf)]

    @functools.partial(
        pl.kernel,
        out_type=jax.ShapeDtypeStruct((K, D), dt),
        mesh=mesh,
        scratch_types=scratch,
    )
    def gk(table_hbm, idx_hbm, out_hbm, idx_v, *rest):
        bufs = rest[:nbuf]
        gsems = rest[nbuf:2 * nbuf]
        osems = rest[2 * nbuf:]
        c = lax.axis_index("c")
        s = lax.axis_index("s")

        def pipeline(base, nch):
            kw = nch * chunk
            pltpu.sync_copy(idx_hbm.at[pl.ds(base, kw)],
                            idx_v.at[pl.ds(0, kw)])

            def g(j):
                sl = j % nbuf
                return pltpu.async_copy(
                    table_hbm.at[idx_v.at[pl.ds(j * chunk, chunk)]],
                    bufs[sl], gsems[sl])

            gcp = [None] * nch
            ocp = [None] * nch
            depth = max(1, min(nbuf - 1, nch))
            for j in range(depth):
                gcp[j] = g(j)
            for i in range(nch):
                gcp[i].wait()
                ocp[i] = pltpu.async_copy(
                    bufs[i % nbuf], out_hbm.at[pl.ds(base + i * chunk, chunk)],
                    osems[i % nbuf])
                j = i + depth
                if j < nch:
                    if j - nbuf >= 0:
                        ocp[j - nbuf].wait()
                    gcp[j] = g(j)
            for i in range(max(0, nch - nbuf), nch):
                ocp[i].wait()

        if n0 > 0:
            @pl.when(c == 0)
            def _():
                pipeline(pl.multiple_of(s * (n0 * chunk), 8), n0)
        if n1 > 0:
            @pl.when(c == 1)
            def _():
                pipeline(pl.multiple_of(NS * n0 * chunk + s * (n1 * chunk), 8),
                         n1)

    return gk(table, idx)


def _tc_prep(E, ccat, WE_all, Wc_all, b_all):
    Bn = 2048
    grid = N_PAD // Bn

    def body(e_ref, c_ref, we_ref, wc_ref, b_ref, nzrh_ref, no_ref):
        nall = (jnp.dot(e_ref[...], we_ref[...], preferred_element_type=jnp.float32)
                + jnp.dot(c_ref[...], wc_ref[...], preferred_element_type=jnp.float32)
                + b_ref[...])
        nzrh_ref[...] = nall[:, :3 * H]
        no_ref[...] = nall[:, 3 * H:]

    return pl.pallas_call(
        body,
        grid=(grid,),
        in_specs=[
            pl.BlockSpec((Bn, H), lambda i: (i, 0)),
            pl.BlockSpec((Bn, 16), lambda i: (i, 0)),
            pl.BlockSpec((H, 4 * H), lambda i: (0, 0)),
            pl.BlockSpec((16, 4 * H), lambda i: (0, 0)),
            pl.BlockSpec((1, 4 * H), lambda i: (0, 0)),
        ],
        out_specs=[
            pl.BlockSpec((Bn, 3 * H), lambda i: (i, 0)),
            pl.BlockSpec((Bn, H), lambda i: (i, 0)),
        ],
        out_shape=[
            jax.ShapeDtypeStruct((N_PAD, 3 * H), jnp.float32),
            jax.ShapeDtypeStruct((N_PAD, H), jnp.float32),
        ],
    )(E, ccat, WE_all, Wc_all, b_all)


def _tc_update0(xzrh):
    Bm = 2048
    grid = M_PAD // Bm

    def body(x_ref, h_ref):
        h = jax.nn.sigmoid(x_ref[:, 0:H]) * jnp.tanh(x_ref[:, 2 * H:3 * H])
        h_ref[...] = h
        @pl.when(pl.program_id(0) == 0)
        def _():
            h_ref[0:1, :] = jnp.zeros((1, H), jnp.float32)

    return pl.pallas_call(
        body,
        grid=(grid,),
        in_specs=[pl.BlockSpec((Bm, 3 * H), lambda i: (i, 0))],
        out_specs=pl.BlockSpec((Bm, H), lambda i: (i, 0)),
        out_shape=jax.ShapeDtypeStruct((M_PAD, H), jnp.float32),
    )(xzrh)


def _tc_update(h_nei, xzrh, Ur_t, Ur_b2, Wzh_t, Whh_t):
    Bm = 512
    grid = M_PAD // Bm

    def body(hn_ref, x_ref, ur_ref, urb_ref, wz_ref, wh_ref, h_ref):
        hn = hn_ref[...]
        rp = jnp.dot(hn, ur_ref[...],
                     preferred_element_type=jnp.float32) + urb_ref[...]
        hn3 = hn.reshape(Bm, DEG, H)
        sum_h = jnp.sum(hn3, axis=1)
        xr = x_ref[:, H:2 * H]
        r = jax.nn.sigmoid(rp.reshape(Bm, DEG, H) + xr[:, None, :])
        sum_g = jnp.sum(r * hn3, axis=1)
        z = jax.nn.sigmoid(
            x_ref[:, 0:H]
            + jnp.dot(sum_h, wz_ref[...], preferred_element_type=jnp.float32))
        pre = jnp.tanh(
            x_ref[:, 2 * H:3 * H]
            + jnp.dot(sum_g, wh_ref[...], preferred_element_type=jnp.float32))
        h_ref[...] = (1.0 - z) * sum_h + z * pre
        @pl.when(pl.program_id(0) == 0)
        def _():
            h_ref[0:1, :] = jnp.zeros((1, H), jnp.float32)

    return pl.pallas_call(
        body,
        grid=(grid,),
        in_specs=[
            pl.BlockSpec((Bm * DEG, H), lambda i: (i, 0)),
            pl.BlockSpec((Bm, 3 * H), lambda i: (i, 0)),
            pl.BlockSpec((H, H), lambda i: (0, 0)),
            pl.BlockSpec((1, H), lambda i: (0, 0)),
            pl.BlockSpec((H, H), lambda i: (0, 0)),
            pl.BlockSpec((H, H), lambda i: (0, 0)),
        ],
        out_specs=pl.BlockSpec((Bm, H), lambda i: (i, 0)),
        out_shape=jax.ShapeDtypeStruct((M_PAD, H), jnp.float32),
    )(h_nei, xzrh, Ur_t, Ur_b2, Wzh_t, Whh_t)


def _tc_final(mess_nei, no_node, Wo2_t):
    Bn = 1024
    grid = N_PAD // Bn

    def body(mn_ref, no_ref, w_ref, nv_ref):
        agg = jnp.sum(mn_ref[...].reshape(Bn, DEG, H), axis=1)
        nv_ref[...] = jnp.maximum(
            no_ref[...]
            + jnp.dot(agg, w_ref[...], preferred_element_type=jnp.float32), 0.0)

    return pl.pallas_call(
        body,
        grid=(grid,),
        in_specs=[
            pl.BlockSpec((Bn * DEG, H), lambda i: (i, 0)),
            pl.BlockSpec((Bn, H), lambda i: (i, 0)),
            pl.BlockSpec((H, H), lambda i: (0, 0)),
        ],
        out_specs=pl.BlockSpec((Bn, H), lambda i: (i, 0)),
        out_shape=jax.ShapeDtypeStruct((N_PAD, H), jnp.float32),
    )(mess_nei, no_node, Wo2_t)


def kernel(fnode, fmess, node_graph, mess_graph, scope, cond_lnKD,
           cond_SelectPtoM, embedding, Wz_w, Wz_b, Wr_w, Ur_w, Ur_b,
           Wh_w, Wh_b, out_w, out_b):
    f32 = jnp.float32
    i32 = jnp.int32

    WE_all = jnp.concatenate(
        [Wz_w[:, :H].T, Wr_w[:, :H].T, Wh_w[:, :H].T, out_w[:, :H].T], axis=1)

    def cpart(W):
        return jnp.concatenate(
            [W[:, H:IN].T, jnp.zeros((4, H), f32)], axis=0)

    Wc_all = jnp.concatenate(
        [cpart(Wz_w), cpart(Wr_w), cpart(Wh_w), cpart(out_w)], axis=1)
    b_all = jnp.concatenate(
        [Wz_b, jnp.zeros((H,), f32), Wh_b, out_b])[None, :]
    Wzh_t = Wz_w[:, IN:].T
    Whh_t = Wh_w[:, IN:].T
    Wo2_t = out_w[:, IN:].T
    Ur_t = Ur_w.T
    Ur_b2 = Ur_b[None, :]

    def pad1(a, n):
        return jnp.concatenate([a.astype(i32), jnp.zeros((n - a.shape[0],), i32)])

    fnode_p = pad1(fnode, N_PAD)
    fmess_p = pad1(fmess, M_PAD)
    mg_flat = pad1(mess_graph.reshape(-1), M_PAD * DEG)
    ng_flat = pad1(node_graph.reshape(-1), N_PAD * DEG)
    scope_p = pad1(scope[:, 0], NW * 8)
    ccat = jnp.concatenate(
        [cond_lnKD, cond_SelectPtoM, jnp.zeros((N, 4), f32)], axis=1)
    ccat = jnp.concatenate([ccat, jnp.zeros((N_PAD - N, 16), f32)], axis=0)

    E = _sc_gather(embedding, fnode_p, 64)
    nzrh, no_node = _tc_prep(E, ccat, WE_all, Wc_all, b_all)
    xzrh = _sc_gather(nzrh, fmess_p, 64)
    h = _tc_update0(xzrh)
    gcfg = [(1.0, 0), (1.0, 1), (0.9, 0), (0.8, 0)]
    for i in range(DEPTH - 1):
        hf, hc = gcfg[i]
        hn = _sc_gather(h, mg_flat, 128,
                        heavy_frac=hf, heavy_core=hc)
        h = _tc_update(hn, xzrh, Ur_t, Ur_b2, Wzh_t, Whh_t)
    mn = _sc_gather(h, ng_flat, 128)
    nv = _tc_final(mn, no_node, Wo2_t)
    tv = _sc_gather(nv, scope_p, 8)[:B]
    return tv, h[:M]


# speedup vs baseline: 1.0472x; 1.0472x over previous
import functools

import jax
import jax.numpy as jnp
from jax import lax
from jax.experimental import pallas as pl
from jax.experimental.pallas import tpu as pltpu
from jax.experimental.pallas import tpu_sc as plsc

N = 10000
M = 20000
DEG = 8
H = 128
CK = 8
CS = 4
DEPTH = 5
B = 64
IN = H + CK + CS

NC, NS = 2, 16
NW = NC * NS
N_PAD = 10240
M_PAD = 20480


def _sc_gather(table, idx, chunk, nbuf=4, heavy_frac=0.5, heavy_core=0):
    (K,) = idx.shape
    T, D = table.shape
    dt = table.dtype
    cs = K // (NS * chunk)
    assert K % (NS * chunk) == 0 and chunk % 8 == 0 and chunk <= 128
    a_heavy = max(1, min(cs, round(cs * heavy_frac)))
    a_light = cs - a_heavy
    n0, n1 = ((a_heavy, a_light) if heavy_core == 0
              else (a_light, a_heavy))
    nbuf = min(nbuf, max(n0, n1))

    mesh = plsc.VectorSubcoreMesh(core_axis_name="c", subcore_axis_name="s",
                                  num_cores=NC, num_subcores=NS)
    scratch = [pltpu.VMEM((max(n0, n1) * chunk,), jnp.int32)]
    scratch += [pltpu.VMEM((chunk, D), dt) for _ in range(nbuf)]
    scratch += [pltpu.SemaphoreType.DMA for _ in range(2 * nbuf)]

    @functools.partial(
        pl.kernel,
        out_type=jax.ShapeDtypeStruct((K, D), dt),
        mesh=mesh,
        scratch_types=scratch,
    )
    def gk(table_hbm, idx_hbm, out_hbm, idx_v, *rest):
        bufs = rest[:nbuf]
        gsems = rest[nbuf:2 * nbuf]
        osems = rest[2 * nbuf:]
        c = lax.axis_index("c")
        s = lax.axis_index("s")

        def pipeline(base, nch):
            kw = nch * chunk
            pltpu.sync_copy(idx_hbm.at[pl.ds(base, kw)],
                            idx_v.at[pl.ds(0, kw)])

            def g(j):
                sl = j % nbuf
                return pltpu.async_copy(
                    table_hbm.at[idx_v.at[pl.ds(j * chunk, chunk)]],
                    bufs[sl], gsems[sl])

            gcp = [None] * nch
            ocp = [None] * nch
            depth = max(1, min(nbuf - 1, nch))
            for j in range(depth):
                gcp[j] = g(j)
            for i in range(nch):
                gcp[i].wait()
                ocp[i] = pltpu.async_copy(
                    bufs[i % nbuf], out_hbm.at[pl.ds(base + i * chunk, chunk)],
                    osems[i % nbuf])
                j = i + depth
                if j < nch:
                    if j - nbuf >= 0:
                        ocp[j - nbuf].wait()
                    gcp[j] = g(j)
            for i in range(max(0, nch - nbuf), nch):
                ocp[i].wait()

        if n0 > 0:
            @pl.when(c == 0)
            def _():
                pipeline(pl.multiple_of(s * (n0 * chunk), 8), n0)
        if n1 > 0:
            @pl.when(c == 1)
            def _():
                pipeline(pl.multiple_of(NS * n0 * chunk + s * (n1 * chunk), 8),
                         n1)

    return gk(table, idx)


def _tc_prep(E, ccat, WE_all, Wc_all, b_all):
    Bn = 2048
    grid = N_PAD // Bn

    def body(e_ref, c_ref, we_ref, wc_ref, b_ref, nzrh_ref, no_ref):
        nall = (jnp.dot(e_ref[...], we_ref[...], preferred_element_type=jnp.float32)
                + jnp.dot(c_ref[...], wc_ref[...], preferred_element_type=jnp.float32)
                + b_ref[...])
        nzrh_ref[...] = nall[:, :3 * H]
        no_ref[...] = nall[:, 3 * H:]

    return pl.pallas_call(
        body,
        grid=(grid,),
        in_specs=[
            pl.BlockSpec((Bn, H), lambda i: (i, 0)),
            pl.BlockSpec((Bn, 16), lambda i: (i, 0)),
            pl.BlockSpec((H, 4 * H), lambda i: (0, 0)),
            pl.BlockSpec((16, 4 * H), lambda i: (0, 0)),
            pl.BlockSpec((1, 4 * H), lambda i: (0, 0)),
        ],
        out_specs=[
            pl.BlockSpec((Bn, 3 * H), lambda i: (i, 0)),
            pl.BlockSpec((Bn, H), lambda i: (i, 0)),
        ],
        out_shape=[
            jax.ShapeDtypeStruct((N_PAD, 3 * H), jnp.float32),
            jax.ShapeDtypeStruct((N_PAD, H), jnp.float32),
        ],
    )(E, ccat, WE_all, Wc_all, b_all)


def _tc_update0(xzrh):
    Bm = 2048
    grid = M_PAD // Bm

    def body(x_ref, h_ref):
        h = jax.nn.sigmoid(x_ref[:, 0:H]) * jnp.tanh(x_ref[:, 2 * H:3 * H])
        h_ref[...] = h
        @pl.when(pl.program_id(0) == 0)
        def _():
            h_ref[0:1, :] = jnp.zeros((1, H), jnp.float32)

    return pl.pallas_call(
        body,
        grid=(grid,),
        in_specs=[pl.BlockSpec((Bm, 3 * H), lambda i: (i, 0))],
        out_specs=pl.BlockSpec((Bm, H), lambda i: (i, 0)),
        out_shape=jax.ShapeDtypeStruct((M_PAD, H), jnp.float32),
    )(xzrh)


def _tc_update(h_nei, xzrh, Ur_t, Ur_b2, Wzh_t, Whh_t):
    Bm = 512
    grid = M_PAD // Bm

    def body(hn_ref, x_ref, ur_ref, urb_ref, wz_ref, wh_ref, h_ref):
        hn = hn_ref[...]
        rp = jnp.dot(hn, ur_ref[...],
                     preferred_element_type=jnp.float32) + urb_ref[...]
        hn3 = hn.reshape(Bm, DEG, H)
        sum_h = jnp.sum(hn3, axis=1)
        xr = x_ref[:, H:2 * H]
        r = jax.nn.sigmoid(rp.reshape(Bm, DEG, H) + xr[:, None, :])
        sum_g = jnp.sum(r * hn3, axis=1)
        z = jax.nn.sigmoid(
            x_ref[:, 0:H]
            + jnp.dot(sum_h, wz_ref[...], preferred_element_type=jnp.float32))
        pre = jnp.tanh(
            x_ref[:, 2 * H:3 * H]
            + jnp.dot(sum_g, wh_ref[...], preferred_element_type=jnp.float32))
        h_ref[...] = (1.0 - z) * sum_h + z * pre
        @pl.when(pl.program_id(0) == 0)
        def _():
            h_ref[0:1, :] = jnp.zeros((1, H), jnp.float32)

    return pl.pallas_call(
        body,
        grid=(grid,),
        in_specs=[
            pl.BlockSpec((Bm * DEG, H), lambda i: (i, 0)),
            pl.BlockSpec((Bm, 3 * H), lambda i: (i, 0)),
            pl.BlockSpec((H, H), lambda i: (0, 0)),
            pl.BlockSpec((1, H), lambda i: (0, 0)),
            pl.BlockSpec((H, H), lambda i: (0, 0)),
            pl.BlockSpec((H, H), lambda i: (0, 0)),
        ],
        out_specs=pl.BlockSpec((Bm, H), lambda i: (i, 0)),
        out_shape=jax.ShapeDtypeStruct((M_PAD, H), jnp.float32),
    )(h_nei, xzrh, Ur_t, Ur_b2, Wzh_t, Whh_t)


def _tc_final(mess_nei, no_node, Wo2_t):
    Bn = 1024
    grid = N_PAD // Bn

    def body(mn_ref, no_ref, w_ref, nv_ref):
        agg = jnp.sum(mn_ref[...].reshape(Bn, DEG, H), axis=1)
        nv_ref[...] = jnp.maximum(
            no_ref[...]
            + jnp.dot(agg, w_ref[...], preferred_element_type=jnp.float32), 0.0)

    return pl.pallas_call(
        body,
        grid=(grid,),
        in_specs=[
            pl.BlockSpec((Bn * DEG, H), lambda i: (i, 0)),
            pl.BlockSpec((Bn, H), lambda i: (i, 0)),
            pl.BlockSpec((H, H), lambda i: (0, 0)),
        ],
        out_specs=pl.BlockSpec((Bn, H), lambda i: (i, 0)),
        out_shape=jax.ShapeDtypeStruct((N_PAD, H), jnp.float32),
    )(mess_nei, no_node, Wo2_t)


def kernel(fnode, fmess, node_graph, mess_graph, scope, cond_lnKD,
           cond_SelectPtoM, embedding, Wz_w, Wz_b, Wr_w, Ur_w, Ur_b,
           Wh_w, Wh_b, out_w, out_b):
    f32 = jnp.float32
    i32 = jnp.int32

    WE_all = jnp.concatenate(
        [Wz_w[:, :H].T, Wr_w[:, :H].T, Wh_w[:, :H].T, out_w[:, :H].T], axis=1)

    def cpart(W):
        return jnp.concatenate(
            [W[:, H:IN].T, jnp.zeros((4, H), f32)], axis=0)

    Wc_all = jnp.concatenate(
        [cpart(Wz_w), cpart(Wr_w), cpart(Wh_w), cpart(out_w)], axis=1)
    b_all = jnp.concatenate(
        [Wz_b, jnp.zeros((H,), f32), Wh_b, out_b])[None, :]
    Wzh_t = Wz_w[:, IN:].T
    Whh_t = Wh_w[:, IN:].T
    Wo2_t = out_w[:, IN:].T
    Ur_t = Ur_w.T
    Ur_b2 = Ur_b[None, :]

    def pad1(a, n):
        return jnp.concatenate([a.astype(i32), jnp.zeros((n - a.shape[0],), i32)])

    fnode_p = pad1(fnode, N_PAD)
    fmess_p = pad1(fmess, M_PAD)
    mg_flat = pad1(mess_graph.reshape(-1), M_PAD * DEG)
    ng_flat = pad1(node_graph.reshape(-1), N_PAD * DEG)
    scope_p = pad1(scope[:, 0], NW * 8)
    ccat = jnp.concatenate(
        [cond_lnKD, cond_SelectPtoM, jnp.zeros((N, 4), f32)], axis=1)
    ccat = jnp.concatenate([ccat, jnp.zeros((N_PAD - N, 16), f32)], axis=0)

    E = _sc_gather(embedding, fnode_p, 64)
    nzrh, no_node = _tc_prep(E, ccat, WE_all, Wc_all, b_all)
    xzrh = _sc_gather(nzrh, fmess_p, 64)
    h = _tc_update0(xzrh)
    for _ in range(DEPTH - 1):
        hn = _sc_gather(h, mg_flat, 128)
        h = _tc_update(hn, xzrh, Ur_t, Ur_b2, Wzh_t, Whh_t)
    mn = _sc_gather(h, ng_flat, 128)
    nv = _tc_final(mn, no_node, Wo2_t)
    tv = _sc_gather(nv, scope_p, 8)[:B]
    return tv, h[:M]


# speedup vs baseline: 1.0631x; 1.0153x over previous
import functools

import jax
import jax.numpy as jnp
from jax import lax
from jax.experimental import pallas as pl
from jax.experimental.pallas import tpu as pltpu
from jax.experimental.pallas import tpu_sc as plsc

N = 10000
M = 20000
DEG = 8
H = 128
CK = 8
CS = 4
DEPTH = 5
B = 64
IN = H + CK + CS

NC, NS = 2, 16
NW = NC * NS
N_PAD = 10240
M_PAD = 20480


def _sc_gather(table, idx, chunk, nbuf=4, heavy_frac=0.5, heavy_core=0):
    (K,) = idx.shape
    T, D = table.shape
    dt = table.dtype
    cs = K // (NS * chunk)
    assert K % (NS * chunk) == 0 and chunk % 8 == 0 and chunk <= 128
    a_heavy = max(1, min(cs, round(cs * heavy_frac)))
    a_light = cs - a_heavy
    n0, n1 = ((a_heavy, a_light) if heavy_core == 0
              else (a_light, a_heavy))
    nbuf = min(nbuf, max(n0, n1))

    mesh = plsc.VectorSubcoreMesh(core_axis_name="c", subcore_axis_name="s",
                                  num_cores=NC, num_subcores=NS)
    scratch = [pltpu.VMEM((max(n0, n1) * chunk,), jnp.int32)]
    scratch += [pltpu.VMEM((chunk, D), dt) for _ in range(nbuf)]
    scratch += [pltpu.SemaphoreType.DMA for _ in range(2 * nbuf)]

    @functools.partial(
        pl.kernel,
        out_type=jax.ShapeDtypeStruct((K, D), dt),
        mesh=mesh,
        scratch_types=scratch,
    )
    def gk(table_hbm, idx_hbm, out_hbm, idx_v, *rest):
        bufs = rest[:nbuf]
        gsems = rest[nbuf:2 * nbuf]
        osems = rest[2 * nbuf:]
        c = lax.axis_index("c")
        s = lax.axis_index("s")

        def pipeline(base, nch):
            kw = nch * chunk
            pltpu.sync_copy(idx_hbm.at[pl.ds(base, kw)],
                            idx_v.at[pl.ds(0, kw)])

            def g(j):
                sl = j % nbuf
                return pltpu.async_copy(
                    table_hbm.at[idx_v.at[pl.ds(j * chunk, chunk)]],
                    bufs[sl], gsems[sl])

            gcp = [None] * nch
            ocp = [None] * nch
            depth = max(1, min(nbuf - 1, nch))
            for j in range(depth):
                gcp[j] = g(j)
            for i in range(nch):
                gcp[i].wait()
                ocp[i] = pltpu.async_copy(
                    bufs[i % nbuf], out_hbm.at[pl.ds(base + i * chunk, chunk)],
                    osems[i % nbuf])
                j = i + depth
                if j < nch:
                    if j - nbuf >= 0:
                        ocp[j - nbuf].wait()
                    gcp[j] = g(j)
            for i in range(max(0, nch - nbuf), nch):
                ocp[i].wait()

        if n0 > 0:
            @pl.when(c == 0)
            def _():
                pipeline(pl.multiple_of(s * (n0 * chunk), 8), n0)
        if n1 > 0:
            @pl.when(c == 1)
            def _():
                pipeline(pl.multiple_of(NS * n0 * chunk + s * (n1 * chunk), 8),
                         n1)

    return gk(table, idx)


def _tc_prep(E, ccat, WE_all, Wc_all, b_all):
    Bn = 2048
    grid = N_PAD // Bn

    def body(e_ref, c_ref, we_ref, wc_ref, b_ref, nzrh_ref, no_ref):
        nall = (jnp.dot(e_ref[...], we_ref[...], preferred_element_type=jnp.float32)
                + jnp.dot(c_ref[...], wc_ref[...], preferred_element_type=jnp.float32)
                + b_ref[...])
        nzrh_ref[...] = nall[:, :3 * H]
        no_ref[...] = nall[:, 3 * H:]

    return pl.pallas_call(
        body,
        grid=(grid,),
        in_specs=[
            pl.BlockSpec((Bn, H), lambda i: (i, 0)),
            pl.BlockSpec((Bn, 16), lambda i: (i, 0)),
            pl.BlockSpec((H, 4 * H), lambda i: (0, 0)),
            pl.BlockSpec((16, 4 * H), lambda i: (0, 0)),
            pl.BlockSpec((1, 4 * H), lambda i: (0, 0)),
        ],
        out_specs=[
            pl.BlockSpec((Bn, 3 * H), lambda i: (i, 0)),
            pl.BlockSpec((Bn, H), lambda i: (i, 0)),
        ],
        out_shape=[
            jax.ShapeDtypeStruct((N_PAD, 3 * H), jnp.float32),
            jax.ShapeDtypeStruct((N_PAD, H), jnp.float32),
        ],
    )(E, ccat, WE_all, Wc_all, b_all)


def _tc_update0(xzrh):
    Bm = 2048
    grid = M_PAD // Bm

    def body(x_ref, h_ref):
        h = jax.nn.sigmoid(x_ref[:, 0:H]) * jnp.tanh(x_ref[:, 2 * H:3 * H])
        h_ref[...] = h
        @pl.when(pl.program_id(0) == 0)
        def _():
            h_ref[0:1, :] = jnp.zeros((1, H), jnp.float32)

    return pl.pallas_call(
        body,
        grid=(grid,),
        in_specs=[pl.BlockSpec((Bm, 3 * H), lambda i: (i, 0))],
        out_specs=pl.BlockSpec((Bm, H), lambda i: (i, 0)),
        out_shape=jax.ShapeDtypeStruct((M_PAD, H), jnp.float32),
    )(xzrh)


def _tc_update(h_nei, xzrh, Ur_t, Ur_b2, Wzh_t, Whh_t):
    Bm = 1024
    grid = M_PAD // Bm

    def body(hn_ref, x_ref, ur_ref, urb_ref, wz_ref, wh_ref, h_ref):
        hn = hn_ref[...]
        rp = jnp.dot(hn, ur_ref[...],
                     preferred_element_type=jnp.float32) + urb_ref[...]
        hn3 = hn.reshape(Bm, DEG, H)
        sum_h = jnp.sum(hn3, axis=1)
        xr = x_ref[:, H:2 * H]
        r = jax.nn.sigmoid(rp.reshape(Bm, DEG, H) + xr[:, None, :])
        sum_g = jnp.sum(r * hn3, axis=1)
        z = jax.nn.sigmoid(
            x_ref[:, 0:H]
            + jnp.dot(sum_h, wz_ref[...], preferred_element_type=jnp.float32))
        pre = jnp.tanh(
            x_ref[:, 2 * H:3 * H]
            + jnp.dot(sum_g, wh_ref[...], preferred_element_type=jnp.float32))
        h_ref[...] = (1.0 - z) * sum_h + z * pre
        @pl.when(pl.program_id(0) == 0)
        def _():
            h_ref[0:1, :] = jnp.zeros((1, H), jnp.float32)

    return pl.pallas_call(
        body,
        grid=(grid,),
        in_specs=[
            pl.BlockSpec((Bm * DEG, H), lambda i: (i, 0)),
            pl.BlockSpec((Bm, 3 * H), lambda i: (i, 0)),
            pl.BlockSpec((H, H), lambda i: (0, 0)),
            pl.BlockSpec((1, H), lambda i: (0, 0)),
            pl.BlockSpec((H, H), lambda i: (0, 0)),
            pl.BlockSpec((H, H), lambda i: (0, 0)),
        ],
        out_specs=pl.BlockSpec((Bm, H), lambda i: (i, 0)),
        out_shape=jax.ShapeDtypeStruct((M_PAD, H), jnp.float32),
    )(h_nei, xzrh, Ur_t, Ur_b2, Wzh_t, Whh_t)


def _tc_final(mess_nei, no_node, Wo2_t):
    Bn = 1024
    grid = N_PAD // Bn

    def body(mn_ref, no_ref, w_ref, nv_ref):
        agg = jnp.sum(mn_ref[...].reshape(Bn, DEG, H), axis=1)
        nv_ref[...] = jnp.maximum(
            no_ref[...]
            + jnp.dot(agg, w_ref[...], preferred_element_type=jnp.float32), 0.0)

    return pl.pallas_call(
        body,
        grid=(grid,),
        in_specs=[
            pl.BlockSpec((Bn * DEG, H), lambda i: (i, 0)),
            pl.BlockSpec((Bn, H), lambda i: (i, 0)),
            pl.BlockSpec((H, H), lambda i: (0, 0)),
        ],
        out_specs=pl.BlockSpec((Bn, H), lambda i: (i, 0)),
        out_shape=jax.ShapeDtypeStruct((N_PAD, H), jnp.float32),
    )(mess_nei, no_node, Wo2_t)


def kernel(fnode, fmess, node_graph, mess_graph, scope, cond_lnKD,
           cond_SelectPtoM, embedding, Wz_w, Wz_b, Wr_w, Ur_w, Ur_b,
           Wh_w, Wh_b, out_w, out_b):
    f32 = jnp.float32
    i32 = jnp.int32

    WE_all = jnp.concatenate(
        [Wz_w[:, :H].T, Wr_w[:, :H].T, Wh_w[:, :H].T, out_w[:, :H].T], axis=1)

    def cpart(W):
        return jnp.concatenate(
            [W[:, H:IN].T, jnp.zeros((4, H), f32)], axis=0)

    Wc_all = jnp.concatenate(
        [cpart(Wz_w), cpart(Wr_w), cpart(Wh_w), cpart(out_w)], axis=1)
    b_all = jnp.concatenate(
        [Wz_b, jnp.zeros((H,), f32), Wh_b, out_b])[None, :]
    Wzh_t = Wz_w[:, IN:].T
    Whh_t = Wh_w[:, IN:].T
    Wo2_t = out_w[:, IN:].T
    Ur_t = Ur_w.T
    Ur_b2 = Ur_b[None, :]

    def pad1(a, n):
        return jnp.concatenate([a.astype(i32), jnp.zeros((n - a.shape[0],), i32)])

    fnode_p = pad1(fnode, N_PAD)
    fmess_p = pad1(fmess, M_PAD)
    mg_flat = pad1(mess_graph.reshape(-1), M_PAD * DEG)
    ng_flat = pad1(node_graph.reshape(-1), N_PAD * DEG)
    scope_p = pad1(scope[:, 0], NW * 8)
    ccat = jnp.concatenate(
        [cond_lnKD, cond_SelectPtoM, jnp.zeros((N, 4), f32)], axis=1)
    ccat = jnp.concatenate([ccat, jnp.zeros((N_PAD - N, 16), f32)], axis=0)

    E = _sc_gather(embedding, fnode_p, 64)
    nzrh, no_node = _tc_prep(E, ccat, WE_all, Wc_all, b_all)
    xzrh = _sc_gather(nzrh, fmess_p, 64)
    h = _tc_update0(xzrh)
    for _ in range(DEPTH - 1):
        hn = _sc_gather(h, mg_flat, 128)
        h = _tc_update(hn, xzrh, Ur_t, Ur_b2, Wzh_t, Whh_t)
    mn = _sc_gather(h, ng_flat, 128)
    nv = _tc_final(mn, no_node, Wo2_t)
    tv = _sc_gather(nv, scope_p, 8)[:B]
    return tv, h[:M]


# speedup vs baseline: 1.0657x; 1.0024x over previous
import functools

import jax
import jax.numpy as jnp
from jax import lax
from jax.experimental import pallas as pl
from jax.experimental.pallas import tpu as pltpu
from jax.experimental.pallas import tpu_sc as plsc

N = 10000
M = 20000
DEG = 8
H = 128
CK = 8
CS = 4
DEPTH = 5
B = 64
IN = H + CK + CS

NC, NS = 2, 16
NW = NC * NS
N_PAD = 10240
M_PAD = 20480


def _sc_gather(table, idx, chunk, nbuf=4, heavy_frac=0.5, heavy_core=0):
    (K,) = idx.shape
    T, D = table.shape
    dt = table.dtype
    cs = K // (NS * chunk)
    assert K % (NS * chunk) == 0 and chunk % 8 == 0 and chunk <= 128
    a_heavy = max(1, min(cs, round(cs * heavy_frac)))
    a_light = cs - a_heavy
    n0, n1 = ((a_heavy, a_light) if heavy_core == 0
              else (a_light, a_heavy))
    nbuf = min(nbuf, max(n0, n1))

    mesh = plsc.VectorSubcoreMesh(core_axis_name="c", subcore_axis_name="s",
                                  num_cores=NC, num_subcores=NS)
    scratch = [pltpu.VMEM((max(n0, n1) * chunk,), jnp.int32)]
    scratch += [pltpu.VMEM((chunk, D), dt) for _ in range(nbuf)]
    scratch += [pltpu.SemaphoreType.DMA for _ in range(2 * nbuf)]

    @functools.partial(
        pl.kernel,
        out_type=jax.ShapeDtypeStruct((K, D), dt),
        mesh=mesh,
        scratch_types=scratch,
    )
    def gk(table_hbm, idx_hbm, out_hbm, idx_v, *rest):
        bufs = rest[:nbuf]
        gsems = rest[nbuf:2 * nbuf]
        osems = rest[2 * nbuf:]
        c = lax.axis_index("c")
        s = lax.axis_index("s")

        def pipeline(base, nch):
            kw = nch * chunk
            pltpu.sync_copy(idx_hbm.at[pl.ds(base, kw)],
                            idx_v.at[pl.ds(0, kw)])

            def g(j):
                sl = j % nbuf
                return pltpu.async_copy(
                    table_hbm.at[idx_v.at[pl.ds(j * chunk, chunk)]],
                    bufs[sl], gsems[sl])

            gcp = [None] * nch
            ocp = [None] * nch
            depth = max(1, min(nbuf - 1, nch))
            for j in range(depth):
                gcp[j] = g(j)
            for i in range(nch):
                gcp[i].wait()
                ocp[i] = pltpu.async_copy(
                    bufs[i % nbuf], out_hbm.at[pl.ds(base + i * chunk, chunk)],
                    osems[i % nbuf])
                j = i + depth
                if j < nch:
                    if j - nbuf >= 0:
                        ocp[j - nbuf].wait()
                    gcp[j] = g(j)
            for i in range(max(0, nch - nbuf), nch):
                ocp[i].wait()

        if n0 > 0:
            @pl.when(c == 0)
            def _():
                pipeline(pl.multiple_of(s * (n0 * chunk), 8), n0)
        if n1 > 0:
            @pl.when(c == 1)
            def _():
                pipeline(pl.multiple_of(NS * n0 * chunk + s * (n1 * chunk), 8),
                         n1)

    return gk(table, idx)


def _tc_prep(E, ccat, WE_all, Wc_all, b_all):
    Bn = 2048
    grid = N_PAD // Bn

    def body(e_ref, c_ref, we_ref, wc_ref, b_ref, nzrh_ref, no_ref):
        nall = (jnp.dot(e_ref[...], we_ref[...], preferred_element_type=jnp.float32)
                + jnp.dot(c_ref[...], wc_ref[...], preferred_element_type=jnp.float32)
                + b_ref[...])
        nzrh_ref[...] = nall[:, :3 * H]
        no_ref[...] = nall[:, 3 * H:]

    return pl.pallas_call(
        body,
        grid=(grid,),
        in_specs=[
            pl.BlockSpec((Bn, H), lambda i: (i, 0)),
            pl.BlockSpec((Bn, 16), lambda i: (i, 0)),
            pl.BlockSpec((H, 4 * H), lambda i: (0, 0)),
            pl.BlockSpec((16, 4 * H), lambda i: (0, 0)),
            pl.BlockSpec((1, 4 * H), lambda i: (0, 0)),
        ],
        out_specs=[
            pl.BlockSpec((Bn, 3 * H), lambda i: (i, 0)),
            pl.BlockSpec((Bn, H), lambda i: (i, 0)),
        ],
        out_shape=[
            jax.ShapeDtypeStruct((N_PAD, 3 * H), jnp.float32),
            jax.ShapeDtypeStruct((N_PAD, H), jnp.float32),
        ],
    )(E, ccat, WE_all, Wc_all, b_all)


def _tc_update0(xzrh):
    Bm = 2048
    grid = M_PAD // Bm

    def body(x_ref, h_ref):
        h = jax.nn.sigmoid(x_ref[:, 0:H]) * jnp.tanh(x_ref[:, 2 * H:3 * H])
        h_ref[...] = h
        @pl.when(pl.program_id(0) == 0)
        def _():
            h_ref[0:1, :] = jnp.zeros((1, H), jnp.float32)

    return pl.pallas_call(
        body,
        grid=(grid,),
        in_specs=[pl.BlockSpec((Bm, 3 * H), lambda i: (i, 0))],
        out_specs=pl.BlockSpec((Bm, H), lambda i: (i, 0)),
        out_shape=jax.ShapeDtypeStruct((M_PAD, H), jnp.float32),
    )(xzrh)


def _tc_update(h_nei, xzrh, Ur_t, Ur_b2, Wzh_t, Whh_t):
    Bm = 2048
    grid = M_PAD // Bm

    def body(hn_ref, x_ref, ur_ref, urb_ref, wz_ref, wh_ref, h_ref):
        hn = hn_ref[...]
        rp = jnp.dot(hn, ur_ref[...],
                     preferred_element_type=jnp.float32) + urb_ref[...]
        hn3 = hn.reshape(Bm, DEG, H)
        sum_h = jnp.sum(hn3, axis=1)
        xr = x_ref[:, H:2 * H]
        r = jax.nn.sigmoid(rp.reshape(Bm, DEG, H) + xr[:, None, :])
        sum_g = jnp.sum(r * hn3, axis=1)
        z = jax.nn.sigmoid(
            x_ref[:, 0:H]
            + jnp.dot(sum_h, wz_ref[...], preferred_element_type=jnp.float32))
        pre = jnp.tanh(
            x_ref[:, 2 * H:3 * H]
            + jnp.dot(sum_g, wh_ref[...], preferred_element_type=jnp.float32))
        h_ref[...] = (1.0 - z) * sum_h + z * pre
        @pl.when(pl.program_id(0) == 0)
        def _():
            h_ref[0:1, :] = jnp.zeros((1, H), jnp.float32)

    return pl.pallas_call(
        body,
        grid=(grid,),
        in_specs=[
            pl.BlockSpec((Bm * DEG, H), lambda i: (i, 0)),
            pl.BlockSpec((Bm, 3 * H), lambda i: (i, 0)),
            pl.BlockSpec((H, H), lambda i: (0, 0)),
            pl.BlockSpec((1, H), lambda i: (0, 0)),
            pl.BlockSpec((H, H), lambda i: (0, 0)),
            pl.BlockSpec((H, H), lambda i: (0, 0)),
        ],
        out_specs=pl.BlockSpec((Bm, H), lambda i: (i, 0)),
        out_shape=jax.ShapeDtypeStruct((M_PAD, H), jnp.float32),
    )(h_nei, xzrh, Ur_t, Ur_b2, Wzh_t, Whh_t)


def _tc_final(mess_nei, no_node, Wo2_t):
    Bn = 1024
    grid = N_PAD // Bn

    def body(mn_ref, no_ref, w_ref, nv_ref):
        agg = jnp.sum(mn_ref[...].reshape(Bn, DEG, H), axis=1)
        nv_ref[...] = jnp.maximum(
            no_ref[...]
            + jnp.dot(agg, w_ref[...], preferred_element_type=jnp.float32), 0.0)

    return pl.pallas_call(
        body,
        grid=(grid,),
        in_specs=[
            pl.BlockSpec((Bn * DEG, H), lambda i: (i, 0)),
            pl.BlockSpec((Bn, H), lambda i: (i, 0)),
            pl.BlockSpec((H, H), lambda i: (0, 0)),
        ],
        out_specs=pl.BlockSpec((Bn, H), lambda i: (i, 0)),
        out_shape=jax.ShapeDtypeStruct((N_PAD, H), jnp.float32),
    )(mess_nei, no_node, Wo2_t)


def kernel(fnode, fmess, node_graph, mess_graph, scope, cond_lnKD,
           cond_SelectPtoM, embedding, Wz_w, Wz_b, Wr_w, Ur_w, Ur_b,
           Wh_w, Wh_b, out_w, out_b):
    f32 = jnp.float32
    i32 = jnp.int32

    WE_all = jnp.concatenate(
        [Wz_w[:, :H].T, Wr_w[:, :H].T, Wh_w[:, :H].T, out_w[:, :H].T], axis=1)

    def cpart(W):
        return jnp.concatenate(
            [W[:, H:IN].T, jnp.zeros((4, H), f32)], axis=0)

    Wc_all = jnp.concatenate(
        [cpart(Wz_w), cpart(Wr_w), cpart(Wh_w), cpart(out_w)], axis=1)
    b_all = jnp.concatenate(
        [Wz_b, jnp.zeros((H,), f32), Wh_b, out_b])[None, :]
    Wzh_t = Wz_w[:, IN:].T
    Whh_t = Wh_w[:, IN:].T
    Wo2_t = out_w[:, IN:].T
    Ur_t = Ur_w.T
    Ur_b2 = Ur_b[None, :]

    def pad1(a, n):
        return jnp.concatenate([a.astype(i32), jnp.zeros((n - a.shape[0],), i32)])

    fnode_p = pad1(fnode, N_PAD)
    fmess_p = pad1(fmess, M_PAD)
    mg_flat = pad1(mess_graph.reshape(-1), M_PAD * DEG)
    ng_flat = pad1(node_graph.reshape(-1), N_PAD * DEG)
    scope_p = pad1(scope[:, 0], NW * 8)
    ccat = jnp.concatenate(
        [cond_lnKD, cond_SelectPtoM, jnp.zeros((N, 4), f32)], axis=1)
    ccat = jnp.concatenate([ccat, jnp.zeros((N_PAD - N, 16), f32)], axis=0)

    E = _sc_gather(embedding, fnode_p, 64)
    nzrh, no_node = _tc_prep(E, ccat, WE_all, Wc_all, b_all)
    xzrh = _sc_gather(nzrh, fmess_p, 64)
    h = _tc_update0(xzrh)
    for _ in range(DEPTH - 1):
        hn = _sc_gather(h, mg_flat, 128)
        h = _tc_update(hn, xzrh, Ur_t, Ur_b2, Wzh_t, Whh_t)
    mn = _sc_gather(h, ng_flat, 128)
    nv = _tc_final(mn, no_node, Wo2_t)
    tv = _sc_gather(nv, scope_p, 8)[:B]
    return tv, h[:M]


# speedup vs baseline: 1.0683x; 1.0024x over previous
import functools

import jax
import jax.numpy as jnp
from jax import lax
from jax.experimental import pallas as pl
from jax.experimental.pallas import tpu as pltpu
from jax.experimental.pallas import tpu_sc as plsc

N = 10000
M = 20000
DEG = 8
H = 128
CK = 8
CS = 4
DEPTH = 5
B = 64
IN = H + CK + CS

NC, NS = 2, 16
NW = NC * NS
N_PAD = 10240
M_PAD = 20480


def _sc_gather(table, idx, chunk, nbuf=4, heavy_frac=0.5, heavy_core=0):
    (K,) = idx.shape
    T, D = table.shape
    dt = table.dtype
    cs = K // (NS * chunk)
    assert K % (NS * chunk) == 0 and chunk % 8 == 0 and chunk <= 128
    a_heavy = max(1, min(cs, round(cs * heavy_frac)))
    a_light = cs - a_heavy
    n0, n1 = ((a_heavy, a_light) if heavy_core == 0
              else (a_light, a_heavy))
    nbuf = min(nbuf, max(n0, n1))

    mesh = plsc.VectorSubcoreMesh(core_axis_name="c", subcore_axis_name="s",
                                  num_cores=NC, num_subcores=NS)
    scratch = [pltpu.VMEM((max(n0, n1) * chunk,), jnp.int32)]
    scratch += [pltpu.VMEM((chunk, D), dt) for _ in range(nbuf)]
    scratch += [pltpu.SemaphoreType.DMA for _ in range(2 * nbuf)]

    @functools.partial(
        pl.kernel,
        out_type=jax.ShapeDtypeStruct((K, D), dt),
        mesh=mesh,
        scratch_types=scratch,
    )
    def gk(table_hbm, idx_hbm, out_hbm, idx_v, *rest):
        bufs = rest[:nbuf]
        gsems = rest[nbuf:2 * nbuf]
        osems = rest[2 * nbuf:]
        c = lax.axis_index("c")
        s = lax.axis_index("s")

        def pipeline(base, nch):
            kw = nch * chunk
            pltpu.sync_copy(idx_hbm.at[pl.ds(base, kw)],
                            idx_v.at[pl.ds(0, kw)])

            def g(j):
                sl = j % nbuf
                return pltpu.async_copy(
                    table_hbm.at[idx_v.at[pl.ds(j * chunk, chunk)]],
                    bufs[sl], gsems[sl])

            gcp = [None] * nch
            ocp = [None] * nch
            depth = max(1, min(nbuf - 1, nch))
            for j in range(depth):
                gcp[j] = g(j)
            for i in range(nch):
                gcp[i].wait()
                ocp[i] = pltpu.async_copy(
                    bufs[i % nbuf], out_hbm.at[pl.ds(base + i * chunk, chunk)],
                    osems[i % nbuf])
                j = i + depth
                if j < nch:
                    if j - nbuf >= 0:
                        ocp[j - nbuf].wait()
                    gcp[j] = g(j)
            for i in range(max(0, nch - nbuf), nch):
                ocp[i].wait()

        if n0 > 0:
            @pl.when(c == 0)
            def _():
                pipeline(pl.multiple_of(s * (n0 * chunk), 8), n0)
        if n1 > 0:
            @pl.when(c == 1)
            def _():
                pipeline(pl.multiple_of(NS * n0 * chunk + s * (n1 * chunk), 8),
                         n1)

    return gk(table, idx)


def _tc_prep(E, ccat, WE_all, Wc_all, b_all):
    Bn = 2048
    grid = N_PAD // Bn

    def body(e_ref, c_ref, we_ref, wc_ref, b_ref, nzrh_ref, no_ref):
        nall = (jnp.dot(e_ref[...], we_ref[...], preferred_element_type=jnp.float32)
                + jnp.dot(c_ref[...], wc_ref[...], preferred_element_type=jnp.float32)
                + b_ref[...])
        nzrh_ref[...] = nall[:, :3 * H]
        no_ref[...] = nall[:, 3 * H:]

    return pl.pallas_call(
        body,
        grid=(grid,),
        in_specs=[
            pl.BlockSpec((Bn, H), lambda i: (i, 0)),
            pl.BlockSpec((Bn, 16), lambda i: (i, 0)),
            pl.BlockSpec((H, 4 * H), lambda i: (0, 0)),
            pl.BlockSpec((16, 4 * H), lambda i: (0, 0)),
            pl.BlockSpec((1, 4 * H), lambda i: (0, 0)),
        ],
        out_specs=[
            pl.BlockSpec((Bn, 3 * H), lambda i: (i, 0)),
            pl.BlockSpec((Bn, H), lambda i: (i, 0)),
        ],
        out_shape=[
            jax.ShapeDtypeStruct((N_PAD, 3 * H), jnp.float32),
            jax.ShapeDtypeStruct((N_PAD, H), jnp.float32),
        ],
    )(E, ccat, WE_all, Wc_all, b_all)


def _tc_update0(xzrh):
    Bm = 2048
    grid = M_PAD // Bm

    def body(x_ref, h_ref):
        h = jax.nn.sigmoid(x_ref[:, 0:H]) * jnp.tanh(x_ref[:, 2 * H:3 * H])
        h_ref[...] = h
        @pl.when(pl.program_id(0) == 0)
        def _():
            h_ref[0:1, :] = jnp.zeros((1, H), jnp.float32)

    return pl.pallas_call(
        body,
        grid=(grid,),
        in_specs=[pl.BlockSpec((Bm, 3 * H), lambda i: (i, 0))],
        out_specs=pl.BlockSpec((Bm, H), lambda i: (i, 0)),
        out_shape=jax.ShapeDtypeStruct((M_PAD, H), jnp.float32),
    )(xzrh)


def _tc_update(h_nei, xzrh, Ur_t, Ur_b2, Wzh_t, Whh_t):
    Bm = 2048
    grid = M_PAD // Bm

    def body(hn_ref, x_ref, ur_ref, urb_ref, wz_ref, wh_ref, h_ref):
        hn = hn_ref[...]
        rp = jnp.dot(hn, ur_ref[...],
                     preferred_element_type=jnp.float32) + urb_ref[...]
        hn3 = hn.reshape(Bm, DEG, H)
        sum_h = jnp.sum(hn3, axis=1)
        xr = x_ref[:, H:2 * H]
        r = jax.nn.sigmoid(rp.reshape(Bm, DEG, H) + xr[:, None, :])
        sum_g = jnp.sum(r * hn3, axis=1)
        z = jax.nn.sigmoid(
            x_ref[:, 0:H]
            + jnp.dot(sum_h, wz_ref[...], preferred_element_type=jnp.float32))
        pre = jnp.tanh(
            x_ref[:, 2 * H:3 * H]
            + jnp.dot(sum_g, wh_ref[...], preferred_element_type=jnp.float32))
        h_ref[...] = (1.0 - z) * sum_h + z * pre
        @pl.when(pl.program_id(0) == 0)
        def _():
            h_ref[0:1, :] = jnp.zeros((1, H), jnp.float32)

    return pl.pallas_call(
        body,
        grid=(grid,),
        in_specs=[
            pl.BlockSpec((Bm * DEG, H), lambda i: (i, 0)),
            pl.BlockSpec((Bm, 3 * H), lambda i: (i, 0)),
            pl.BlockSpec((H, H), lambda i: (0, 0)),
            pl.BlockSpec((1, H), lambda i: (0, 0)),
            pl.BlockSpec((H, H), lambda i: (0, 0)),
            pl.BlockSpec((H, H), lambda i: (0, 0)),
        ],
        out_specs=pl.BlockSpec((Bm, H), lambda i: (i, 0)),
        out_shape=jax.ShapeDtypeStruct((M_PAD, H), jnp.float32),
    )(h_nei, xzrh, Ur_t, Ur_b2, Wzh_t, Whh_t)


def _tc_final(mess_nei, no_node, Wo2_t):
    Bn = 2048
    grid = N_PAD // Bn

    def body(mn_ref, no_ref, w_ref, nv_ref):
        agg = jnp.sum(mn_ref[...].reshape(Bn, DEG, H), axis=1)
        nv_ref[...] = jnp.maximum(
            no_ref[...]
            + jnp.dot(agg, w_ref[...], preferred_element_type=jnp.float32), 0.0)

    return pl.pallas_call(
        body,
        grid=(grid,),
        in_specs=[
            pl.BlockSpec((Bn * DEG, H), lambda i: (i, 0)),
            pl.BlockSpec((Bn, H), lambda i: (i, 0)),
            pl.BlockSpec((H, H), lambda i: (0, 0)),
        ],
        out_specs=pl.BlockSpec((Bn, H), lambda i: (i, 0)),
        out_shape=jax.ShapeDtypeStruct((N_PAD, H), jnp.float32),
    )(mess_nei, no_node, Wo2_t)


def kernel(fnode, fmess, node_graph, mess_graph, scope, cond_lnKD,
           cond_SelectPtoM, embedding, Wz_w, Wz_b, Wr_w, Ur_w, Ur_b,
           Wh_w, Wh_b, out_w, out_b):
    f32 = jnp.float32
    i32 = jnp.int32

    WE_all = jnp.concatenate(
        [Wz_w[:, :H].T, Wr_w[:, :H].T, Wh_w[:, :H].T, out_w[:, :H].T], axis=1)

    def cpart(W):
        return jnp.concatenate(
            [W[:, H:IN].T, jnp.zeros((4, H), f32)], axis=0)

    Wc_all = jnp.concatenate(
        [cpart(Wz_w), cpart(Wr_w), cpart(Wh_w), cpart(out_w)], axis=1)
    b_all = jnp.concatenate(
        [Wz_b, jnp.zeros((H,), f32), Wh_b, out_b])[None, :]
    Wzh_t = Wz_w[:, IN:].T
    Whh_t = Wh_w[:, IN:].T
    Wo2_t = out_w[:, IN:].T
    Ur_t = Ur_w.T
    Ur_b2 = Ur_b[None, :]

    def pad1(a, n):
        return jnp.concatenate([a.astype(i32), jnp.zeros((n - a.shape[0],), i32)])

    fnode_p = pad1(fnode, N_PAD)
    fmess_p = pad1(fmess, M_PAD)
    mg_flat = pad1(mess_graph.reshape(-1), M_PAD * DEG)
    ng_flat = pad1(node_graph.reshape(-1), N_PAD * DEG)
    scope_p = pad1(scope[:, 0], NW * 8)
    ccat = jnp.concatenate(
        [cond_lnKD, cond_SelectPtoM, jnp.zeros((N, 4), f32)], axis=1)
    ccat = jnp.concatenate([ccat, jnp.zeros((N_PAD - N, 16), f32)], axis=0)

    E = _sc_gather(embedding, fnode_p, 64)
    nzrh, no_node = _tc_prep(E, ccat, WE_all, Wc_all, b_all)
    xzrh = _sc_gather(nzrh, fmess_p, 64)
    h = _tc_update0(xzrh)
    for _ in range(DEPTH - 1):
        hn = _sc_gather(h, mg_flat, 128)
        h = _tc_update(hn, xzrh, Ur_t, Ur_b2, Wzh_t, Whh_t)
    mn = _sc_gather(h, ng_flat, 128)
    nv = _tc_final(mn, no_node, Wo2_t)
    tv = _sc_gather(nv, scope_p, 8)[:B]
    return tv, h[:M]
